# Initial kernel scaffold; baseline (speedup 1.0000x reference)
import jax
import jax.numpy as jnp
from jax.experimental import pallas as pl


def kernel(x, W_gen, b_gen, W_sol, b_sol, seg_ids):
    raise NotImplementedError("write your pallas kernel here")


# speedup vs baseline: 3.0297x; 3.0297x over previous
import functools

import jax
import jax.numpy as jnp
from jax import lax
from jax.experimental import pallas as pl
from jax.experimental.pallas import tpu as pltpu
from jax.experimental.pallas import tpu_sc as plsc

_P = 800000
_C = 50000
_B = 16
_CP = 51200
_NW = 32
_RPB = 128
_SB = 10
_STAGE = _RPB * _SB
_NSTB = _P // _STAGE
_TCK = 3200
_ZCH = _CP // 16
_RW = _CP // _NW

_SC_PARAMS = pltpu.CompilerParams(use_tc_tiling_on_sc=False)
_SC_MESH = dict(core_axis_name="c", subcore_axis_name="s")


def _worker_id():
    return lax.axis_index("s") * 2 + lax.axis_index("c")


def _worker_blocks(wid):
    nb = jnp.where(wid < 17, 20, 19).astype(jnp.int32)
    b0 = (wid * 19 + jnp.minimum(wid, 17)).astype(jnp.int32)
    return b0, nb


def _mm_body(x_ref, wg_ref, bg_ref, ws_ref, bs_ref, inner_ref, yt_ref, acc):
    i = pl.program_id(0)

    @pl.when(i == 0)
    def _():
        inner = (
            jnp.dot(x_ref[...], wg_ref[...], preferred_element_type=jnp.float32)
            + bg_ref[...]
        )
        acc[...] = inner
        inner_ref[...] = inner

    inner = acc[...]
    y = jnp.dot(inner, ws_ref[...], preferred_element_type=jnp.float32)
    y = jnp.maximum(y + bs_ref[...], 0.0) + 1e-16
    eye = jnp.eye(_B, dtype=jnp.float32)
    yt_ref[...] = lax.dot_general(
        y, eye, (((0,), (0,)), ((), ())), preferred_element_type=jnp.float32
    )


def _tc_matmul(x, W_gen, b_gen, W_sol, b_sol):
    return pl.pallas_call(
        _mm_body,
        grid=(_P // _TCK,),
        in_specs=[
            pl.BlockSpec((_B, 1024), lambda i: (0, 0)),
            pl.BlockSpec((1024, 64), lambda i: (0, 0)),
            pl.BlockSpec((1, 64), lambda i: (0, 0)),
            pl.BlockSpec((64, _TCK), lambda i: (0, i)),
            pl.BlockSpec((1, _TCK), lambda i: (0, i)),
        ],
        out_specs=[
            pl.BlockSpec((_B, 64), lambda i: (0, 0)),
            pl.BlockSpec((_TCK, _B), lambda i: (i, 0)),
        ],
        out_shape=[
            jax.ShapeDtypeStruct((_B, 64), jnp.float32),
            jax.ShapeDtypeStruct((_P, _B), jnp.float32),
        ],
        scratch_shapes=[pltpu.VMEM((_B, 64), jnp.float32)],
    )(x, W_gen, b_gen.reshape(1, 64), W_sol, b_sol.reshape(1, _P))


@functools.partial(
    pl.kernel,
    out_type=jax.ShapeDtypeStruct((2, _CP, _B), jnp.float32),
    mesh=plsc.VectorSubcoreMesh(**_SC_MESH),
    scratch_types=[
        pltpu.VMEM((_STAGE, _B), jnp.float32),
        pltpu.VMEM((_SB, _RPB), jnp.int32),
        pltpu.VMEM_SHARED((_CP, _B), jnp.float32),
        pltpu.SemaphoreType.DMA,
    ],
    compiler_params=_SC_PARAMS,
)
def _sc_segsum(y_hbm, seg_hbm, zeros_hbm, part_hbm, yv, sv, totals, sem):
    cid = lax.axis_index("c")
    sid = lax.axis_index("s")
    wid = _worker_id()
    pltpu.sync_copy(
        zeros_hbm.at[pl.ds(sid * _ZCH, _ZCH)], totals.at[pl.ds(sid * _ZCH, _ZCH)]
    )
    plsc.subcore_barrier()
    b0, nb = _worker_blocks(wid)

    def stage(j, carry):
        blk = b0 + j
        pltpu.sync_copy(y_hbm.at[pl.ds(blk * _STAGE, _STAGE)], yv)
        pltpu.sync_copy(seg_hbm.at[pl.ds(blk * _SB, _SB)], sv)
        descs = [
            pltpu.async_copy(
                yv.at[pl.ds(k * _RPB, _RPB)], totals.at[sv.at[k]], sem, add=True
            )
            for k in range(_SB)
        ]
        for d in descs:
            d.wait()
        return carry

    lax.fori_loop(0, nb, stage, jnp.int32(0))
    plsc.subcore_barrier()
    pltpu.sync_copy(
        totals.at[pl.ds(sid * _ZCH, _ZCH)],
        part_hbm.at[cid].at[pl.ds(sid * _ZCH, _ZCH)],
    )


@functools.partial(
    pl.kernel,
    out_type=jax.ShapeDtypeStruct((_CP, _B), jnp.float32),
    mesh=plsc.VectorSubcoreMesh(**_SC_MESH),
    scratch_types=[
        pltpu.VMEM((_RW, _B), jnp.float32),
        pltpu.VMEM((_RW, _B), jnp.float32),
        pltpu.VMEM((_RW, _B), jnp.float32),
    ],
    compiler_params=_SC_PARAMS,
)
def _sc_inv(part_hbm, inv_hbm, p0v, p1v, ov):
    wid = _worker_id()
    base = wid * _RW
    pltpu.sync_copy(part_hbm.at[0].at[pl.ds(base, _RW)], p0v)
    pltpu.sync_copy(part_hbm.at[1].at[pl.ds(base, _RW)], p1v)

    def row(r, carry):
        t = p0v[r] + p1v[r]
        ov[r] = jnp.float32(1.0) / t
        return carry

    lax.fori_loop(0, _RW, row, jnp.int32(0))
    pltpu.sync_copy(ov, inv_hbm.at[pl.ds(base, _RW)])


@functools.partial(
    pl.kernel,
    out_type=jax.ShapeDtypeStruct((_P, _B), jnp.float32),
    mesh=plsc.VectorSubcoreMesh(**_SC_MESH),
    scratch_types=[
        pltpu.VMEM((_SB, _RPB), jnp.int32),
        pltpu.VMEM((_STAGE, _B), jnp.float32),
        pltpu.SemaphoreType.DMA,
    ],
    compiler_params=_SC_PARAMS,
)
def _sc_gather(seg_hbm, inv_hbm, g_hbm, sv, gv, sem):
    wid = _worker_id()
    b0, nb = _worker_blocks(wid)

    def stage(j, carry):
        blk = b0 + j
        pltpu.sync_copy(seg_hbm.at[pl.ds(blk * _SB, _SB)], sv)
        descs = [
            pltpu.async_copy(
                inv_hbm.at[sv.at[k]], gv.at[pl.ds(k * _RPB, _RPB)], sem
            )
            for k in range(_SB)
        ]
        for d in descs:
            d.wait()
        pltpu.sync_copy(gv, g_hbm.at[pl.ds(blk * _STAGE, _STAGE)])
        return carry

    lax.fori_loop(0, nb, stage, jnp.int32(0))


def _fin_body(yt_ref, g_ref, out_ref):
    prod = yt_ref[...] * g_ref[...]
    eye = jnp.eye(_B, dtype=jnp.float32)
    out_ref[...] = lax.dot_general(
        eye, prod, (((1,), (1,)), ((), ())), preferred_element_type=jnp.float32
    )


def _tc_finalize(y_t, g):
    return pl.pallas_call(
        _fin_body,
        grid=(_P // _TCK,),
        in_specs=[
            pl.BlockSpec((_TCK, _B), lambda i: (i, 0)),
            pl.BlockSpec((_TCK, _B), lambda i: (i, 0)),
        ],
        out_specs=pl.BlockSpec((_B, _TCK), lambda i: (0, i)),
        out_shape=jax.ShapeDtypeStruct((_B, _P), jnp.float32),
    )(y_t, g)


def kernel(x, W_gen, b_gen, W_sol, b_sol, seg_ids):
    seg2 = seg_ids.astype(jnp.int32).reshape(_P // _RPB, _RPB)
    zeros = jnp.zeros((_CP, _B), jnp.float32)
    inner, y_t = _tc_matmul(x, W_gen, b_gen, W_sol, b_sol)
    part = _sc_segsum(y_t, seg2, zeros)
    inv = _sc_inv(part)
    g = _sc_gather(seg2, inv)
    out = _tc_finalize(y_t, g)
    return out, inner


# speedup vs baseline: 6.3092x; 2.0824x over previous
import functools

import jax
import jax.numpy as jnp
from jax import lax
from jax.experimental import pallas as pl
from jax.experimental.pallas import tpu as pltpu
from jax.experimental.pallas import tpu_sc as plsc

_P = 800000
_C = 50000
_B = 16
_CP = 51200
_NW = 32
_TCK = 6400
_GRP = _TCK // 8
_PR = _P // 8
_IPO = 100
_SPR = 100
_SPP = _SPR * 8
_NST = _P // _SPP
_ZCH = _CP // 16
_RW = _CP // _NW

_SC_PARAMS = pltpu.CompilerParams(use_tc_tiling_on_sc=False)
_SC_MESH = dict(core_axis_name="c", subcore_axis_name="s")


def _worker_id():
    return lax.axis_index("s") * 2 + lax.axis_index("c")


def _worker_stages(wid):
    nb = jnp.where(wid < 8, 32, 31).astype(jnp.int32)
    b0 = (wid * 31 + jnp.minimum(wid, 8)).astype(jnp.int32)
    return b0, nb


def _eye():
    return jnp.eye(_B, dtype=jnp.float32)


def _pack(y):
    yt = lax.dot_general(
        y, _eye(), (((0,), (0,)), ((), ())), preferred_element_type=jnp.float32
    )
    y3 = yt.reshape(8, _GRP, _B)
    return jnp.concatenate([y3[a] for a in range(8)], axis=1)


def _unpack(p8):
    pieces = [p8[:, 16 * a : 16 * a + 16] for a in range(8)]
    pt = jnp.concatenate(pieces, axis=0)
    return lax.dot_general(
        _eye(), pt, (((1,), (1,)), ((), ())), preferred_element_type=jnp.float32
    )


def _mm_body(x_ref, wg_ref, bg_ref, ws_ref, bs_ref, inner_ref, y8_ref, acc):
    i = pl.program_id(0)

    @pl.when(i == 0)
    def _():
        inner = (
            jnp.dot(x_ref[...], wg_ref[...], preferred_element_type=jnp.float32)
            + bg_ref[...]
        )
        acc[...] = inner
        inner_ref[...] = inner

    inner = acc[...]
    y = jnp.dot(inner, ws_ref[...], preferred_element_type=jnp.float32)
    y = jnp.maximum(y + bs_ref[...], 0.0) + 1e-16
    y8_ref[...] = _pack(y)


def _tc_matmul(x, W_gen, b_gen, W_sol, b_sol):
    return pl.pallas_call(
        _mm_body,
        grid=(_P // _TCK,),
        in_specs=[
            pl.BlockSpec((_B, 1024), lambda i: (0, 0)),
            pl.BlockSpec((1024, 64), lambda i: (0, 0)),
            pl.BlockSpec((1, 64), lambda i: (0, 0)),
            pl.BlockSpec((64, _TCK), lambda i: (0, i)),
            pl.BlockSpec((1, _TCK), lambda i: (0, i)),
        ],
        out_specs=[
            pl.BlockSpec((_B, 64), lambda i: (0, 0)),
            pl.BlockSpec((_GRP, 128), lambda i: (i, 0)),
        ],
        out_shape=[
            jax.ShapeDtypeStruct((_B, 64), jnp.float32),
            jax.ShapeDtypeStruct((_PR, 128), jnp.float32),
        ],
        scratch_shapes=[pltpu.VMEM((_B, 64), jnp.float32)],
    )(x, W_gen, b_gen.reshape(1, 64), W_sol, b_sol.reshape(1, _P))


def _seg_slice(seg_hbm, s):
    return seg_hbm.at[pl.ds(8 * s, 8)]


@functools.partial(
    pl.kernel,
    out_type=jax.ShapeDtypeStruct((2, _CP, _B), jnp.float32),
    mesh=plsc.VectorSubcoreMesh(**_SC_MESH),
    scratch_types=[
        pltpu.VMEM((_SPP, _B), jnp.float32),
        pltpu.VMEM((8, _IPO), jnp.int32),
        pltpu.VMEM((_ZCH, _B), jnp.float32),
        pltpu.VMEM_SHARED((_CP, _B), jnp.float32),
        pltpu.SemaphoreType.DMA,
        pltpu.SemaphoreType.DMA,
    ],
    compiler_params=_SC_PARAMS,
)
def _sc_segsum(y8_hbm, seg_hbm, part_hbm, yv, sv, zv, totals, sem_l, sem_a):
    cid = lax.axis_index("c")
    sid = lax.axis_index("s")
    wid = _worker_id()

    def zrow(r, carry):
        zv[r] = jnp.zeros((_B,), jnp.float32)
        return carry

    lax.fori_loop(0, _ZCH, zrow, jnp.int32(0))
    pltpu.sync_copy(zv, totals.at[pl.ds(sid * _ZCH, _ZCH)])
    plsc.subcore_barrier()
    b0, nb = _worker_stages(wid)

    def stage(j, carry):
        s = b0 + j
        row0 = s * _SPR
        loads = [
            pltpu.async_copy(
                y8_hbm.at[pl.ds(row0, _SPR), pl.ds(16 * a, 16)],
                yv.at[pl.ds(_IPO * a, _IPO)],
                sem_l,
            )
            for a in range(8)
        ]
        pltpu.sync_copy(_seg_slice(seg_hbm, s), sv)
        for d in loads:
            d.wait()
        adds = [
            pltpu.async_copy(
                yv.at[pl.ds(_IPO * a, _IPO)], totals.at[sv.at[a]], sem_a, add=True
            )
            for a in range(8)
        ]
        for d in adds:
            d.wait()
        return carry

    lax.fori_loop(0, nb, stage, jnp.int32(0))
    plsc.subcore_barrier()
    pltpu.sync_copy(
        totals.at[pl.ds(sid * _ZCH, _ZCH)],
        part_hbm.at[cid].at[pl.ds(sid * _ZCH, _ZCH)],
    )


@functools.partial(
    pl.kernel,
    out_type=jax.ShapeDtypeStruct((_CP, _B), jnp.float32),
    mesh=plsc.VectorSubcoreMesh(**_SC_MESH),
    scratch_types=[
        pltpu.VMEM((_RW, _B), jnp.float32),
        pltpu.VMEM((_RW, _B), jnp.float32),
        pltpu.VMEM((_RW, _B), jnp.float32),
    ],
    compiler_params=_SC_PARAMS,
)
def _sc_inv(part_hbm, inv_hbm, p0v, p1v, ov):
    wid = _worker_id()
    base = wid * _RW
    pltpu.sync_copy(part_hbm.at[0].at[pl.ds(base, _RW)], p0v)
    pltpu.sync_copy(part_hbm.at[1].at[pl.ds(base, _RW)], p1v)

    def row(r, carry):
        t = p0v[r] + p1v[r]
        ov[r] = jnp.float32(1.0) / t
        return carry

    lax.fori_loop(0, _RW, row, jnp.int32(0))
    pltpu.sync_copy(ov, inv_hbm.at[pl.ds(base, _RW)])


@functools.partial(
    pl.kernel,
    out_type=jax.ShapeDtypeStruct((_PR, 128), jnp.float32),
    mesh=plsc.VectorSubcoreMesh(**_SC_MESH),
    scratch_types=[
        pltpu.VMEM((8, _IPO), jnp.int32),
        pltpu.VMEM((_SPP, _B), jnp.float32),
        pltpu.SemaphoreType.DMA,
        pltpu.SemaphoreType.DMA,
    ],
    compiler_params=_SC_PARAMS,
)
def _sc_gather(seg_hbm, inv_hbm, g8_hbm, sv, gv, sem_g, sem_w):
    wid = _worker_id()
    b0, nb = _worker_stages(wid)

    def stage(j, carry):
        s = b0 + j
        row0 = s * _SPR
        pltpu.sync_copy(_seg_slice(seg_hbm, s), sv)
        gets = [
            pltpu.async_copy(
                inv_hbm.at[sv.at[a]], gv.at[pl.ds(_IPO * a, _IPO)], sem_g
            )
            for a in range(8)
        ]
        for d in gets:
            d.wait()
        puts = [
            pltpu.async_copy(
                gv.at[pl.ds(_IPO * a, _IPO)],
                g8_hbm.at[pl.ds(row0, _SPR), pl.ds(16 * a, 16)],
                sem_w,
            )
            for a in range(8)
        ]
        for d in puts:
            d.wait()
        return carry

    lax.fori_loop(0, nb, stage, jnp.int32(0))


def _fin_body(y8_ref, g8_ref, out_ref):
    prod = y8_ref[...] * g8_ref[...]
    out_ref[...] = _unpack(prod)


def _tc_finalize(y8, g8):
    return pl.pallas_call(
        _fin_body,
        grid=(_P // _TCK,),
        in_specs=[
            pl.BlockSpec((_GRP, 128), lambda i: (i, 0)),
            pl.BlockSpec((_GRP, 128), lambda i: (i, 0)),
        ],
        out_specs=pl.BlockSpec((_B, _TCK), lambda i: (0, i)),
        out_shape=jax.ShapeDtypeStruct((_B, _P), jnp.float32),
    )(y8, g8)


def kernel(x, W_gen, b_gen, W_sol, b_sol, seg_ids):
    seg4 = (
        seg_ids.astype(jnp.int32)
        .reshape(_P // _TCK, 8, 8, _IPO)
        .transpose(0, 2, 1, 3)
        .reshape(_NST * 8, _IPO)
    )
    inner, y8 = _tc_matmul(x, W_gen, b_gen, W_sol, b_sol)
    part = _sc_segsum(y8, seg4)
    inv = _sc_inv(part)
    g8 = _sc_gather(seg4, inv)
    out = _tc_finalize(y8, g8)
    return out, inner
